# baseline (speedup 1.0000x reference)
import functools

import jax
import jax.numpy as jnp
from jax import lax
from jax.experimental import pallas as pl
from jax.experimental.pallas import tpu as pltpu
from jax.experimental.pallas import tpu_sc as plsc

_B, _S, _D = 1, 2048, 768
_E, _K, _FFN = 8, 2, 1536
_T = _B * _S
_BLK = 256
_NB = (_K * _T + _E * (_BLK - 1) + _BLK - 1) // _BLK
_P = _NB * _BLK
_NW = 32
_TPW = _T // _NW
_SPW = _P // _NW
_CH = 32


def _router_body(x_ref, rw_ref, rb_ref,
                 pos1_ref, pos2_ref, g1_ref, g2_ref, blke_ref, nblk_ref,
                 xcp_ref):
    xb = x_ref[...]
    xcp_ref[...] = xb
    logits = lax.dot_general(
        xb, rw_ref[...], (((1,), (1,)), ((), ())),
        preferred_element_type=jnp.float32) + rb_ref[...]
    m = jnp.max(logits, axis=-1, keepdims=True)
    p = jnp.exp(logits - m)
    p = p / jnp.sum(p, axis=-1, keepdims=True)
    ii = lax.broadcasted_iota(jnp.int32, (_T, _E), 1)
    v1 = jnp.max(p, axis=-1, keepdims=True)
    i1 = jnp.min(jnp.where(p == v1, ii, _E), axis=-1, keepdims=True)
    m1 = (ii == i1).astype(jnp.float32)
    pm = jnp.where(ii == i1, -1.0, p)
    v2 = jnp.max(pm, axis=-1, keepdims=True)
    i2 = jnp.min(jnp.where(pm == v2, ii, _E), axis=-1, keepdims=True)
    m2 = (ii == i2).astype(jnp.float32)
    s = v1 + v2 + 1e-9
    g1_ref[...] = v1 / s
    g2_ref[...] = v2 / s

    ri = lax.broadcasted_iota(jnp.int32, (_T, _T), 0)
    ci = lax.broadcasted_iota(jnp.int32, (_T, _T), 1)
    tril = (ci <= ri).astype(jnp.bfloat16)
    m12 = jnp.concatenate([m1, m2], axis=1).astype(jnp.bfloat16)
    c12 = lax.dot_general(tril, m12, (((1,), (0,)), ((), ())),
                          preferred_element_type=jnp.float32)
    c1 = c12[:, :_E]
    c2 = c12[:, _E:]
    counts1 = c1[_T - 1:_T, :]
    counts = counts1 + c2[_T - 1:_T, :]
    aligned = jnp.floor((counts + (_BLK - 1)) * (1.0 / _BLK)) * _BLK
    er = lax.broadcasted_iota(jnp.int32, (_E, _E), 0)
    ec = lax.broadcasted_iota(jnp.int32, (_E, _E), 1)
    off = lax.dot_general(aligned, (er < ec).astype(jnp.float32),
                          (((1,), (0,)), ((), ())),
                          preferred_element_type=jnp.float32)
    pos1 = jnp.sum(m1 * (off + c1 - 1.0), axis=-1, keepdims=True)
    pos2 = jnp.sum(m2 * (off + counts1 + c2 - 1.0), axis=-1, keepdims=True)
    pos1_ref[...] = pos1.astype(jnp.int32)
    pos2_ref[...] = pos2.astype(jnp.int32)

    ends_row = off + aligned
    eye = (lax.broadcasted_iota(jnp.int32, (_E, _E), 0) ==
           lax.broadcasted_iota(jnp.int32, (_E, _E), 1)).astype(jnp.float32)
    ends_col = jnp.sum(ends_row * eye, axis=-1, keepdims=True)
    bbase = lax.broadcasted_iota(
        jnp.int32, (_E, _NB), 1).astype(jnp.float32) * _BLK
    blke = jnp.sum((bbase >= ends_col).astype(jnp.float32), axis=0,
                   keepdims=True)
    blke_ref[...] = jnp.minimum(blke, _E - 1.0).astype(jnp.int32)
    nblk_ref[...] = (jnp.sum(aligned, axis=-1, keepdims=True)
                     * (1.0 / _BLK)).astype(jnp.int32)


def _slots_body(p1_ref, p2_ref, g1_ref, g2_ref, src_ref, gs_ref):
    b = pl.program_id(0)
    s = (b * _BLK + lax.broadcasted_iota(jnp.int32, (_BLK, 1), 0))
    m1 = (p1_ref[...] == s).astype(jnp.float32)
    m2 = (p2_ref[...] == s).astype(jnp.float32)
    tok = lax.broadcasted_iota(
        jnp.int32, (_BLK, _T), 1).astype(jnp.float32)
    srcf = jnp.sum((m1 + m2) * tok, axis=-1, keepdims=True)
    src_ref[...] = srcf.astype(jnp.int32)
    gs_ref[...] = jnp.sum(m1 * g1_ref[...] + m2 * g2_ref[...],
                          axis=-1, keepdims=True)


def _expert_body(be_ref, nb_ref, xs_ref, w1_ref, b1_ref, w2_ref, b2_ref,
                 we_ref, gs_ref, ys_ref):
    b = pl.program_id(0)

    @pl.when(b < nb_ref[0])
    def _():
        h = xs_ref[...] + we_ref[0]
        y = lax.dot_general(
            h, w1_ref[0], (((1,), (1,)), ((), ())),
            preferred_element_type=jnp.float32) + b1_ref[0]
        y = y * 0.5 * (1.0 + lax.erf(y * (2.0 ** -0.5)))
        z = lax.dot_general(
            y, w2_ref[0], (((1,), (1,)), ((), ())),
            preferred_element_type=jnp.float32)
        r = h + z + b2_ref[0]
        mu = jnp.mean(r, axis=-1, keepdims=True)
        var = jnp.mean((r - mu) ** 2, axis=-1, keepdims=True)
        eo = (r - mu) * lax.rsqrt(var + 1e-5)
        ys_ref[...] = eo * gs_ref[...]


def _pairsum_body(a_ref, b_ref, out_ref):
    out_ref[...] = a_ref[...] + b_ref[...]


@functools.lru_cache(maxsize=None)
def _sc_kernels():
    mesh = plsc.VectorSubcoreMesh(core_axis_name="c", subcore_axis_name="s")

    _HC = _SPW // 2

    @functools.partial(
        pl.kernel,
        out_type=jax.ShapeDtypeStruct((_P, _D), jnp.float32),
        mesh=mesh,
        name="sc_dispatch_gather",
        scratch_types=[
            pltpu.VMEM((_TPW,), jnp.int32),
            pltpu.VMEM((_TPW,), jnp.int32),
            pltpu.VMEM((_TPW, _D), jnp.float32),
            pltpu.VMEM((_TPW, _D), jnp.float32),
            pltpu.SemaphoreType.DMA,
            pltpu.SemaphoreType.DMA,
            pltpu.SemaphoreType.DMA,
            pltpu.SemaphoreType.DMA,
        ],
    )
    def _dispatch(x_hbm, pos1_hbm, pos2_hbm, xs_hbm, idx_a, idx_b,
                  rows_a, rows_b, sga, sgb, swa, swb):
        wid = lax.axis_index("s") * 2 + lax.axis_index("c")
        base = wid * _TPW
        pltpu.sync_copy(pos1_hbm.at[pl.ds(base, _TPW)], idx_a)
        pltpu.sync_copy(pos2_hbm.at[pl.ds(base, _TPW)], idx_b)
        pltpu.sync_copy(x_hbm.at[pl.ds(base, _TPW)], rows_a)
        wa = pltpu.async_copy(rows_a, xs_hbm.at[idx_a], swa)
        wb = pltpu.async_copy(rows_a, xs_hbm.at[idx_b], swb)
        wa.wait()
        wb.wait()

    @functools.partial(
        pl.kernel,
        out_type=[
            jax.ShapeDtypeStruct((_T, _D), jnp.float32),
            jax.ShapeDtypeStruct((_T, _D), jnp.float32),
        ],
        mesh=mesh,
        name="sc_combine_gather",
        scratch_types=[
            pltpu.VMEM((_TPW,), jnp.int32),
            pltpu.VMEM((_TPW,), jnp.int32),
            pltpu.VMEM((_TPW, _D), jnp.float32),
            pltpu.VMEM((_TPW, _D), jnp.float32),
            pltpu.SemaphoreType.DMA,
            pltpu.SemaphoreType.DMA,
            pltpu.SemaphoreType.DMA,
            pltpu.SemaphoreType.DMA,
        ],
    )
    def _combine(ys_hbm, pos1_hbm, pos2_hbm, out1_hbm, out2_hbm, idx1_v,
                 idx2_v, rows1_v, rows2_v, sg1, sg2, sw1, sw2):
        wid = lax.axis_index("s") * 2 + lax.axis_index("c")
        tbase = wid * _TPW
        pltpu.sync_copy(pos1_hbm.at[pl.ds(tbase, _TPW)], idx1_v)
        pltpu.sync_copy(pos2_hbm.at[pl.ds(tbase, _TPW)], idx2_v)
        g1 = pltpu.async_copy(ys_hbm.at[idx1_v], rows1_v, sg1)
        g2 = pltpu.async_copy(ys_hbm.at[idx2_v], rows2_v, sg2)
        g1.wait()
        w1 = pltpu.async_copy(rows1_v, out1_hbm.at[pl.ds(tbase, _TPW)],
                              sw1)
        g2.wait()
        w2 = pltpu.async_copy(rows2_v, out2_hbm.at[pl.ds(tbase, _TPW)],
                              sw2)
        w1.wait()
        w2.wait()

    return _dispatch, _combine


def kernel(x, router_w, router_b, w1, b1, w2, b2, width_emb):
    flat = x.reshape(_T, _D)
    pos1, pos2, g1, g2, blke, nblk, xcp = pl.pallas_call(
        _router_body,
        in_specs=[
            pl.BlockSpec((_T, _D), lambda: (0, 0)),
            pl.BlockSpec((_E, _D), lambda: (0, 0)),
            pl.BlockSpec((1, _E), lambda: (0, 0)),
        ],
        out_specs=[
            pl.BlockSpec((_T, 1), lambda: (0, 0)),
            pl.BlockSpec((_T, 1), lambda: (0, 0)),
            pl.BlockSpec((_T, 1), lambda: (0, 0)),
            pl.BlockSpec((_T, 1), lambda: (0, 0)),
            pl.BlockSpec((1, _NB), lambda: (0, 0)),
            pl.BlockSpec((1, 1), lambda: (0, 0)),
            pl.BlockSpec((_T, _D), lambda: (0, 0)),
        ],
        out_shape=[
            jax.ShapeDtypeStruct((_T, 1), jnp.int32),
            jax.ShapeDtypeStruct((_T, 1), jnp.int32),
            jax.ShapeDtypeStruct((_T, 1), jnp.float32),
            jax.ShapeDtypeStruct((_T, 1), jnp.float32),
            jax.ShapeDtypeStruct((1, _NB), jnp.int32),
            jax.ShapeDtypeStruct((1, 1), jnp.int32),
            jax.ShapeDtypeStruct((_T, _D), jnp.float32),
        ],
    )(flat, router_w, router_b.reshape(1, _E))

    dispatch_fn, combine_fn = _sc_kernels()
    xs = dispatch_fn(xcp, pos1.reshape(_T), pos2.reshape(_T))

    src, gs = pl.pallas_call(
        _slots_body,
        grid=(_NB,),
        in_specs=[
            pl.BlockSpec((1, _T), lambda b: (0, 0)),
            pl.BlockSpec((1, _T), lambda b: (0, 0)),
            pl.BlockSpec((1, _T), lambda b: (0, 0)),
            pl.BlockSpec((1, _T), lambda b: (0, 0)),
        ],
        out_specs=[
            pl.BlockSpec((_BLK, 1), lambda b: (b, 0)),
            pl.BlockSpec((_BLK, 1), lambda b: (b, 0)),
        ],
        out_shape=[
            jax.ShapeDtypeStruct((_P, 1), jnp.int32),
            jax.ShapeDtypeStruct((_P, 1), jnp.float32),
        ],
    )(pos1.reshape(1, _T), pos2.reshape(1, _T),
      g1.reshape(1, _T), g2.reshape(1, _T))

    ys = pl.pallas_call(
        _expert_body,
        grid_spec=pltpu.PrefetchScalarGridSpec(
            num_scalar_prefetch=2,
            grid=(_NB,),
            in_specs=[
                pl.BlockSpec((_BLK, _D), lambda b, be, nb: (b, 0)),
                pl.BlockSpec((1, _FFN, _D), lambda b, be, nb: (be[b], 0, 0)),
                pl.BlockSpec((1, 1, _FFN), lambda b, be, nb: (be[b], 0, 0)),
                pl.BlockSpec((1, _D, _FFN), lambda b, be, nb: (be[b], 0, 0)),
                pl.BlockSpec((1, 1, _D), lambda b, be, nb: (be[b], 0, 0)),
                pl.BlockSpec((1, 1, _D), lambda b, be, nb: (be[b], 0, 0)),
                pl.BlockSpec((_BLK, 1), lambda b, be, nb: (b, 0)),
            ],
            out_specs=pl.BlockSpec((_BLK, _D), lambda b, be, nb: (b, 0)),
        ),
        out_shape=jax.ShapeDtypeStruct((_P, _D), jnp.float32),
    )(blke.reshape(_NB), nblk.reshape(1), xs, w1,
      b1.reshape(_E, 1, _FFN), w2, b2.reshape(_E, 1, _D),
      width_emb.reshape(_E, 1, _D), gs)

    out1, out2 = combine_fn(ys, pos1.reshape(_T), pos2.reshape(_T))
    out = pl.pallas_call(
        _pairsum_body,
        grid=(_T // 256,),
        in_specs=[
            pl.BlockSpec((256, _D), lambda t: (t, 0)),
            pl.BlockSpec((256, _D), lambda t: (t, 0)),
        ],
        out_specs=pl.BlockSpec((256, _D), lambda t: (t, 0)),
        out_shape=jax.ShapeDtypeStruct((_T, _D), jnp.float32),
    )(out1, out2)
    return out.reshape(x.shape)


# speedup vs baseline: 1.0473x; 1.0473x over previous
import functools

import jax
import jax.numpy as jnp
from jax import lax
from jax.experimental import pallas as pl
from jax.experimental.pallas import tpu as pltpu
from jax.experimental.pallas import tpu_sc as plsc

_B, _S, _D = 1, 2048, 768
_E, _K, _FFN = 8, 2, 1536
_T = _B * _S
_BLK = 256
_NB = (_K * _T + _E * (_BLK - 1) + _BLK - 1) // _BLK
_P = _NB * _BLK
_NW = 32
_TPW = _T // _NW


def _router_body(x_ref, rw_ref, rb_ref,
                 pos1_ref, pos2_ref, g1_ref, g2_ref, blke_ref, nblk_ref,
                 xcp_ref):
    xb = x_ref[...]
    xcp_ref[...] = xb
    logits = lax.dot_general(
        xb, rw_ref[...], (((1,), (1,)), ((), ())),
        preferred_element_type=jnp.float32) + rb_ref[...]
    m = jnp.max(logits, axis=-1, keepdims=True)
    p = jnp.exp(logits - m)
    p = p / jnp.sum(p, axis=-1, keepdims=True)
    ii = lax.broadcasted_iota(jnp.int32, (_T, _E), 1)
    v1 = jnp.max(p, axis=-1, keepdims=True)
    i1 = jnp.min(jnp.where(p == v1, ii, _E), axis=-1, keepdims=True)
    m1 = (ii == i1).astype(jnp.float32)
    pm = jnp.where(ii == i1, -1.0, p)
    v2 = jnp.max(pm, axis=-1, keepdims=True)
    i2 = jnp.min(jnp.where(pm == v2, ii, _E), axis=-1, keepdims=True)
    m2 = (ii == i2).astype(jnp.float32)
    s = v1 + v2 + 1e-9
    g1_ref[...] = v1 / s
    g2_ref[...] = v2 / s

    ri = lax.broadcasted_iota(jnp.int32, (_T, _T), 0)
    ci = lax.broadcasted_iota(jnp.int32, (_T, _T), 1)
    tril = (ci <= ri).astype(jnp.bfloat16)
    m12 = jnp.concatenate([m1, m2], axis=1).astype(jnp.bfloat16)
    c12 = lax.dot_general(tril, m12, (((1,), (0,)), ((), ())),
                          preferred_element_type=jnp.float32)
    c1 = c12[:, :_E]
    c2 = c12[:, _E:]
    counts1 = c1[_T - 1:_T, :]
    counts = counts1 + c2[_T - 1:_T, :]
    aligned = jnp.floor((counts + (_BLK - 1)) * (1.0 / _BLK)) * _BLK
    er = lax.broadcasted_iota(jnp.int32, (_E, _E), 0)
    ec = lax.broadcasted_iota(jnp.int32, (_E, _E), 1)
    off = lax.dot_general(aligned, (er < ec).astype(jnp.float32),
                          (((1,), (0,)), ((), ())),
                          preferred_element_type=jnp.float32)
    pos1 = jnp.sum(m1 * (off + c1 - 1.0), axis=-1, keepdims=True)
    pos2 = jnp.sum(m2 * (off + counts1 + c2 - 1.0), axis=-1, keepdims=True)
    pos1_ref[...] = pos1.astype(jnp.int32)
    pos2_ref[...] = pos2.astype(jnp.int32)

    ends_row = off + aligned
    eye = (lax.broadcasted_iota(jnp.int32, (_E, _E), 0) ==
           lax.broadcasted_iota(jnp.int32, (_E, _E), 1)).astype(jnp.float32)
    ends_col = jnp.sum(ends_row * eye, axis=-1, keepdims=True)
    bbase = lax.broadcasted_iota(
        jnp.int32, (_E, _NB), 1).astype(jnp.float32) * _BLK
    blke = jnp.sum((bbase >= ends_col).astype(jnp.float32), axis=0,
                   keepdims=True)
    blke_ref[...] = jnp.minimum(blke, _E - 1.0).astype(jnp.int32)
    nblk_ref[...] = (jnp.sum(aligned, axis=-1, keepdims=True)
                     * (1.0 / _BLK)).astype(jnp.int32)


def _slots_body(p1_ref, p2_ref, g1_ref, g2_ref, gs_ref):
    b = pl.program_id(0)
    s = (b * _BLK + lax.broadcasted_iota(jnp.int32, (_BLK, 1), 0))
    m1 = (p1_ref[...] == s).astype(jnp.float32)
    m2 = (p2_ref[...] == s).astype(jnp.float32)
    gs_ref[...] = jnp.sum(m1 * g1_ref[...] + m2 * g2_ref[...],
                          axis=-1, keepdims=True)


def _expert_body(be_ref, nb_ref, xs_ref, w1_ref, b1_ref, w2_ref, b2_ref,
                 we_ref, gs_ref, ys_ref):
    b = pl.program_id(0)

    @pl.when(b < nb_ref[0])
    def _():
        h = xs_ref[...] + we_ref[0]
        y = lax.dot_general(
            h, w1_ref[0], (((1,), (1,)), ((), ())),
            preferred_element_type=jnp.float32) + b1_ref[0]
        y = y * 0.5 * (1.0 + lax.erf(y * (2.0 ** -0.5)))
        z = lax.dot_general(
            y, w2_ref[0], (((1,), (1,)), ((), ())),
            preferred_element_type=jnp.float32)
        r = h + z + b2_ref[0]
        mu = jnp.mean(r, axis=-1, keepdims=True)
        var = jnp.mean((r - mu) ** 2, axis=-1, keepdims=True)
        eo = (r - mu) * lax.rsqrt(var + 1e-5)
        ys_ref[...] = eo * gs_ref[...]


def _pairsum_body(a_ref, b_ref, out_ref):
    out_ref[...] = a_ref[...] + b_ref[...]


@functools.lru_cache(maxsize=None)
def _sc_kernels():
    mesh = plsc.VectorSubcoreMesh(core_axis_name="c", subcore_axis_name="s")

    @functools.partial(
        pl.kernel,
        out_type=jax.ShapeDtypeStruct((_P, _D), jnp.float32),
        mesh=mesh,
        name="sc_dispatch_gather",
        scratch_types=[
            pltpu.VMEM((_TPW,), jnp.int32),
            pltpu.VMEM((_TPW,), jnp.int32),
            pltpu.VMEM((_TPW, _D), jnp.float32),
            pltpu.VMEM((_TPW, _D), jnp.float32),
            pltpu.SemaphoreType.DMA,
            pltpu.SemaphoreType.DMA,
            pltpu.SemaphoreType.DMA,
            pltpu.SemaphoreType.DMA,
        ],
    )
    def _dispatch(x_hbm, pos1_hbm, pos2_hbm, xs_hbm, idx_a, idx_b,
                  rows_a, rows_b, sga, sgb, swa, swb):
        wid = lax.axis_index("s") * 2 + lax.axis_index("c")
        base = wid * _TPW
        pltpu.sync_copy(pos1_hbm.at[pl.ds(base, _TPW)], idx_a)
        pltpu.sync_copy(pos2_hbm.at[pl.ds(base, _TPW)], idx_b)
        pltpu.sync_copy(x_hbm.at[pl.ds(base, _TPW)], rows_a)
        wa = pltpu.async_copy(rows_a, xs_hbm.at[idx_a], swa)
        wb = pltpu.async_copy(rows_a, xs_hbm.at[idx_b], swb)
        wa.wait()
        wb.wait()

    @functools.partial(
        pl.kernel,
        out_type=[
            jax.ShapeDtypeStruct((_T, _D), jnp.float32),
            jax.ShapeDtypeStruct((_T, _D), jnp.float32),
        ],
        mesh=mesh,
        name="sc_combine_gather",
        scratch_types=[
            pltpu.VMEM((_TPW,), jnp.int32),
            pltpu.VMEM((_TPW,), jnp.int32),
            pltpu.VMEM((_TPW, _D), jnp.float32),
            pltpu.VMEM((_TPW, _D), jnp.float32),
            pltpu.SemaphoreType.DMA,
            pltpu.SemaphoreType.DMA,
            pltpu.SemaphoreType.DMA,
            pltpu.SemaphoreType.DMA,
        ],
    )
    def _combine(ys_hbm, pos1_hbm, pos2_hbm, out1_hbm, out2_hbm, idx1_v,
                 idx2_v, rows1_v, rows2_v, sg1, sg2, sw1, sw2):
        wid = lax.axis_index("s") * 2 + lax.axis_index("c")
        tbase = wid * _TPW
        pltpu.sync_copy(pos1_hbm.at[pl.ds(tbase, _TPW)], idx1_v)
        pltpu.sync_copy(pos2_hbm.at[pl.ds(tbase, _TPW)], idx2_v)
        g1 = pltpu.async_copy(ys_hbm.at[idx1_v], rows1_v, sg1)
        g2 = pltpu.async_copy(ys_hbm.at[idx2_v], rows2_v, sg2)
        g1.wait()
        w1 = pltpu.async_copy(rows1_v, out1_hbm.at[pl.ds(tbase, _TPW)],
                              sw1)
        g2.wait()
        w2 = pltpu.async_copy(rows2_v, out2_hbm.at[pl.ds(tbase, _TPW)],
                              sw2)
        w1.wait()
        w2.wait()

    return _dispatch, _combine


def kernel(x, router_w, router_b, w1, b1, w2, b2, width_emb):
    flat = x.reshape(_T, _D)
    pos1, pos2, g1, g2, blke, nblk, xcp = pl.pallas_call(
        _router_body,
        in_specs=[
            pl.BlockSpec((_T, _D), lambda: (0, 0)),
            pl.BlockSpec((_E, _D), lambda: (0, 0)),
            pl.BlockSpec((1, _E), lambda: (0, 0)),
        ],
        out_specs=[
            pl.BlockSpec((_T, 1), lambda: (0, 0)),
            pl.BlockSpec((_T, 1), lambda: (0, 0)),
            pl.BlockSpec((_T, 1), lambda: (0, 0)),
            pl.BlockSpec((_T, 1), lambda: (0, 0)),
            pl.BlockSpec((1, _NB), lambda: (0, 0)),
            pl.BlockSpec((1, 1), lambda: (0, 0)),
            pl.BlockSpec((_T, _D), lambda: (0, 0)),
        ],
        out_shape=[
            jax.ShapeDtypeStruct((_T, 1), jnp.int32),
            jax.ShapeDtypeStruct((_T, 1), jnp.int32),
            jax.ShapeDtypeStruct((_T, 1), jnp.float32),
            jax.ShapeDtypeStruct((_T, 1), jnp.float32),
            jax.ShapeDtypeStruct((1, _NB), jnp.int32),
            jax.ShapeDtypeStruct((1, 1), jnp.int32),
            jax.ShapeDtypeStruct((_T, _D), jnp.float32),
        ],
    )(flat, router_w, router_b.reshape(1, _E))

    dispatch_fn, combine_fn = _sc_kernels()
    xs = dispatch_fn(xcp, pos1.reshape(_T), pos2.reshape(_T))

    gs = pl.pallas_call(
        _slots_body,
        grid=(_NB,),
        in_specs=[
            pl.BlockSpec((1, _T), lambda b: (0, 0)),
            pl.BlockSpec((1, _T), lambda b: (0, 0)),
            pl.BlockSpec((1, _T), lambda b: (0, 0)),
            pl.BlockSpec((1, _T), lambda b: (0, 0)),
        ],
        out_specs=pl.BlockSpec((_BLK, 1), lambda b: (b, 0)),
        out_shape=jax.ShapeDtypeStruct((_P, 1), jnp.float32),
    )(pos1.reshape(1, _T), pos2.reshape(1, _T),
      g1.reshape(1, _T), g2.reshape(1, _T))

    ys = pl.pallas_call(
        _expert_body,
        grid_spec=pltpu.PrefetchScalarGridSpec(
            num_scalar_prefetch=2,
            grid=(_NB,),
            in_specs=[
                pl.BlockSpec((_BLK, _D), lambda b, be, nb: (b, 0)),
                pl.BlockSpec((1, _FFN, _D), lambda b, be, nb: (be[b], 0, 0)),
                pl.BlockSpec((1, 1, _FFN), lambda b, be, nb: (be[b], 0, 0)),
                pl.BlockSpec((1, _D, _FFN), lambda b, be, nb: (be[b], 0, 0)),
                pl.BlockSpec((1, 1, _D), lambda b, be, nb: (be[b], 0, 0)),
                pl.BlockSpec((1, 1, _D), lambda b, be, nb: (be[b], 0, 0)),
                pl.BlockSpec((_BLK, 1), lambda b, be, nb: (b, 0)),
            ],
            out_specs=pl.BlockSpec((_BLK, _D), lambda b, be, nb: (b, 0)),
        ),
        out_shape=jax.ShapeDtypeStruct((_P, _D), jnp.float32),
    )(blke.reshape(_NB), nblk.reshape(1), xs, w1,
      b1.reshape(_E, 1, _FFN), w2, b2.reshape(_E, 1, _D),
      width_emb.reshape(_E, 1, _D), gs)

    out1, out2 = combine_fn(ys, pos1.reshape(_T), pos2.reshape(_T))
    out = pl.pallas_call(
        _pairsum_body,
        grid=(_T // 256,),
        in_specs=[
            pl.BlockSpec((256, _D), lambda t: (t, 0)),
            pl.BlockSpec((256, _D), lambda t: (t, 0)),
        ],
        out_specs=pl.BlockSpec((256, _D), lambda t: (t, 0)),
        out_shape=jax.ShapeDtypeStruct((_T, _D), jnp.float32),
    )(out1, out2)
    return out.reshape(x.shape)
